# Initial kernel scaffold; baseline (speedup 1.0000x reference)
import jax
import jax.numpy as jnp
from jax.experimental import pallas as pl


def kernel(user_indices, item_indices, adj_entity, adj_relation, user_emb, entity_emb, relation_emb, W0, b0, W1, b1):
    raise NotImplementedError("write your pallas kernel here")


# speedup vs baseline: 18.6233x; 18.6233x over previous
import functools

import jax
import jax.numpy as jnp
from jax import lax
from jax.experimental import pallas as pl
from jax.experimental.pallas import tpu as pltpu
from jax.experimental.pallas import tpu_sc as plsc

DIM = 32
NN = 16
NR = 32
NW = 32
CH_B = 8


def _sc_kernel_body(
    user_idx_hbm, item_idx_hbm, adj_e_hbm, adj_r_hbm,
    user_emb_hbm, entity_emb_hbm, rel_t_hbm,
    u_out, e0_out, p_out, r0_out, e1_out, agg_out,
    item_v, user_v, ubuf, e0buf, e1ids, r0buf, relt_v, pflat,
    eflat, nbflat, nbr, rel2, ebuf, e1buf, aggbuf, sem,
    *, bpw,
):
    cid = lax.axis_index("c")
    sid = lax.axis_index("s")
    wid = sid * 2 + cid
    base = wid * bpw

    pltpu.sync_copy(item_idx_hbm.at[pl.ds(base, bpw)], item_v)
    pltpu.sync_copy(user_idx_hbm.at[pl.ds(base, bpw)], user_v)
    pltpu.sync_copy(rel_t_hbm, relt_v)
    cps = [
        pltpu.async_copy(user_emb_hbm.at[user_v], ubuf, sem),
        pltpu.async_copy(entity_emb_hbm.at[item_v], e0buf, sem),
        pltpu.async_copy(adj_e_hbm.at[item_v], e1ids, sem),
        pltpu.async_copy(adj_r_hbm.at[item_v], r0buf, sem),
    ]
    for cp in cps:
        cp.wait()
    pltpu.sync_copy(ubuf, u_out.at[pl.ds(base, bpw)])
    pltpu.sync_copy(e0buf, e0_out.at[pl.ds(base, bpw)])
    pltpu.sync_copy(r0buf, r0_out.at[pl.ds(base, bpw)])

    def p_body(b, _):
        plo = jnp.zeros((16,), jnp.float32)
        phi = jnp.zeros((16,), jnp.float32)
        ulo = ubuf[b, 0:16]
        uhi = ubuf[b, 16:32]
        for d in range(DIM):
            ud = ulo[d] if d < 16 else uhi[d - 16]
            plo = plo + ud * relt_v[d, 0:16]
            phi = phi + ud * relt_v[d, 16:32]
        inv = 1.0 / DIM
        pflat[pl.ds(b * NR, 16)] = plo * inv
        pflat[pl.ds(b * NR + 16, 16)] = phi * inv
        return 0
    lax.fori_loop(0, bpw, p_body, 0)
    pltpu.sync_copy(pflat, p_out.at[pl.ds(base * NR, bpw * NR)])

    n_chunks = bpw // CH_B

    nodes = CH_B * NN

    def chunk_body(c, _):
        for bb in range(CH_B):
            eflat[pl.ds(bb * NN, NN)] = e1ids[c * CH_B + bb, 0:16]
        cps = [
            pltpu.async_copy(adj_e_hbm.at[eflat], nbr, sem),
            pltpu.async_copy(adj_r_hbm.at[eflat], rel2, sem),
            pltpu.async_copy(entity_emb_hbm.at[eflat], e1buf, sem),
        ]
        cps[0].wait()
        cps[1].wait()
        cps[2].wait()
        def flat_body(i, _):
            nbflat[pl.ds(i * NN, NN)] = nbr[i, 0:16]
            return 0
        lax.fori_loop(0, nodes, flat_body, 0)
        cps = [
            pltpu.async_copy(
                entity_emb_hbm.at[nbflat.at[pl.ds(j * 128, 128)]],
                ebuf.at[pl.ds(j * 128, 128)], sem)
            for j in range(nodes * NN // 128)
        ]
        for cp in cps:
            cp.wait()

        def node_body(node, _):
            b_loc = c * CH_B + node // NN
            rvec = rel2[node, 0:16]
            score = plsc.load_gather(pflat, [b_loc * NR + rvec])
            e = jnp.exp(score)
            s = jnp.sum(e)
            alo = jnp.zeros((16,), jnp.float32)
            ahi = jnp.zeros((16,), jnp.float32)
            for n in range(NN):
                wn = e[n]
                alo = alo + wn * ebuf[node * NN + n, 0:16]
                ahi = ahi + wn * ebuf[node * NN + n, 16:32]
            sinv = 1.0 / jnp.broadcast_to(s, (16,))
            aggbuf[node, 0:16] = alo * sinv
            aggbuf[node, 16:32] = ahi * sinv
            return 0
        lax.fori_loop(0, nodes, node_body, 0)

        row0 = (base + c * CH_B) * NN
        pltpu.sync_copy(e1buf, e1_out.at[pl.ds(row0, nodes)])
        pltpu.sync_copy(aggbuf, agg_out.at[pl.ds(row0, nodes)])
        return 0

    lax.fori_loop(0, n_chunks, chunk_body, 0)


def _tc_kernel_body(e1_ref, agg_ref, e0_ref, p_ref, r0_ref, u_ref,
                    w0_ref, b0_ref, w1_ref, b1_ref, out_ref):
    r0 = r0_ref[...]
    p = p_ref[...]
    scores = jnp.zeros(r0.shape, jnp.float32)
    for r in range(NR):
        scores = scores + jnp.where(r0 == r, p[:, r:r + 1], 0.0)
    m = jnp.max(scores, axis=1, keepdims=True)
    e = jnp.exp(scores - m)
    w0a = e / jnp.sum(e, axis=1, keepdims=True)

    e1 = e1_ref[...]
    agg = agg_ref[...]
    W0 = w0_ref[...]
    b0 = b0_ref[...]
    W1 = w1_ref[...]
    b1 = b1_ref[...]
    W0a, W0b = W0[:DIM, :], W0[DIM:, :]
    W1a, W1b = W1[:DIM, :], W1[DIM:, :]

    agg_a = jnp.zeros((e1.shape[0], DIM), jnp.float32)
    agg_b = jnp.zeros((e1.shape[0], DIM), jnp.float32)
    for n in range(NN):
        sl = slice(n * DIM, (n + 1) * DIM)
        wn = w0a[:, n:n + 1]
        agg_a = agg_a + wn * e1[:, sl]
        v1 = jnp.maximum(
            jnp.dot(e1[:, sl], W0a, preferred_element_type=jnp.float32)
            + jnp.dot(agg[:, sl], W0b, preferred_element_type=jnp.float32)
            + b0, 0.0)
        agg_b = agg_b + wn * v1
    e0 = e0_ref[...]
    h0 = jnp.maximum(
        jnp.dot(e0, W0a, preferred_element_type=jnp.float32)
        + jnp.dot(agg_a, W0b, preferred_element_type=jnp.float32) + b0, 0.0)
    out = jnp.tanh(
        jnp.dot(h0, W1a, preferred_element_type=jnp.float32)
        + jnp.dot(agg_b, W1b, preferred_element_type=jnp.float32) + b1)
    u = u_ref[...]
    out_ref[...] = jax.nn.sigmoid(jnp.sum(out * u, axis=1))


def kernel(user_indices, item_indices, adj_entity, adj_relation,
           user_emb, entity_emb, relation_emb, W0, b0, W1, b1):
    B = user_indices.shape[0]
    bpw = B // NW
    rel_t = relation_emb.T

    mesh = plsc.VectorSubcoreMesh(core_axis_name="c", subcore_axis_name="s")
    out_type = (
        jax.ShapeDtypeStruct((B, DIM), jnp.float32),
        jax.ShapeDtypeStruct((B, DIM), jnp.float32),
        jax.ShapeDtypeStruct((B * NR,), jnp.float32),
        jax.ShapeDtypeStruct((B, NN), jnp.int32),
        jax.ShapeDtypeStruct((B * NN, DIM), jnp.float32),
        jax.ShapeDtypeStruct((B * NN, DIM), jnp.float32),
    )
    scratch = [
        pltpu.VMEM((bpw,), jnp.int32),
        pltpu.VMEM((bpw,), jnp.int32),
        pltpu.VMEM((bpw, DIM), jnp.float32),
        pltpu.VMEM((bpw, DIM), jnp.float32),
        pltpu.VMEM((bpw, NN), jnp.int32),
        pltpu.VMEM((bpw, NN), jnp.int32),
        pltpu.VMEM((DIM, NR), jnp.float32),
        pltpu.VMEM((bpw * NR,), jnp.float32),
        pltpu.VMEM((CH_B * NN,), jnp.int32),
        pltpu.VMEM((CH_B * NN * NN,), jnp.int32),
        pltpu.VMEM((CH_B * NN, NN), jnp.int32),
        pltpu.VMEM((CH_B * NN, NN), jnp.int32),
        pltpu.VMEM((CH_B * NN * NN, DIM), jnp.float32),
        pltpu.VMEM((CH_B * NN, DIM), jnp.float32),
        pltpu.VMEM((CH_B * NN, DIM), jnp.float32),
        pltpu.SemaphoreType.DMA,
    ]
    sc = pl.kernel(
        functools.partial(_sc_kernel_body, bpw=bpw),
        out_type=out_type, mesh=mesh, scratch_types=scratch,
        compiler_params=pltpu.CompilerParams(
            needs_layout_passes=False, use_tc_tiling_on_sc=False),
    )
    u_g, e0, p, r0, e1, agg1 = sc(
        user_indices, item_indices, adj_entity, adj_relation,
        user_emb, entity_emb, rel_t)

    e1f = e1.reshape(B, NN * DIM)
    aggf = agg1.reshape(B, NN * DIM)
    p = p.reshape(B, NR)
    del agg1, e1
    BB = 256
    grid = (B // BB,)
    out = pl.pallas_call(
        _tc_kernel_body,
        grid=grid,
        in_specs=[
            pl.BlockSpec((BB, NN * DIM), lambda i: (i, 0)),
            pl.BlockSpec((BB, NN * DIM), lambda i: (i, 0)),
            pl.BlockSpec((BB, DIM), lambda i: (i, 0)),
            pl.BlockSpec((BB, NR), lambda i: (i, 0)),
            pl.BlockSpec((BB, NN), lambda i: (i, 0)),
            pl.BlockSpec((BB, DIM), lambda i: (i, 0)),
            pl.BlockSpec((2 * DIM, DIM), lambda i: (0, 0)),
            pl.BlockSpec((1, DIM), lambda i: (0, 0)),
            pl.BlockSpec((2 * DIM, DIM), lambda i: (0, 0)),
            pl.BlockSpec((1, DIM), lambda i: (0, 0)),
        ],
        out_specs=pl.BlockSpec((BB,), lambda i: (i,)),
        out_shape=jax.ShapeDtypeStruct((B,), jnp.float32),
    )(e1f, aggf, e0, p, r0, u_g, W0, b0.reshape(1, DIM), W1,
      b1.reshape(1, DIM))
    return out


# speedup vs baseline: 20.9454x; 1.1247x over previous
import functools

import jax
import jax.numpy as jnp
from jax import lax
from jax.experimental import pallas as pl
from jax.experimental.pallas import tpu as pltpu
from jax.experimental.pallas import tpu_sc as plsc

DIM = 32
NN = 16
NR = 32
NW = 32
CH_B = 8


def _sc_kernel_body(
    user_idx_hbm, item_idx_hbm, adj_e_hbm, adj_r_hbm,
    user_emb_hbm, entity_emb_hbm, rel_t_hbm,
    u_out, e0_out, w0_out, e1_out, agg_out,
    item_v, user_v, ubuf, e0buf, e1ids, r0buf, relt_v, pflat, w0buf,
    eflat, nbflat, nbr, rel2, ebuf, e1buf, aggbuf, sem,
    *, bpw,
):
    cid = lax.axis_index("c")
    sid = lax.axis_index("s")
    wid = sid * 2 + cid
    base = wid * bpw

    pltpu.sync_copy(item_idx_hbm.at[pl.ds(base, bpw)], item_v)
    pltpu.sync_copy(user_idx_hbm.at[pl.ds(base, bpw)], user_v)
    pltpu.sync_copy(rel_t_hbm, relt_v)
    cps = [
        pltpu.async_copy(user_emb_hbm.at[user_v], ubuf, sem),
        pltpu.async_copy(entity_emb_hbm.at[item_v], e0buf, sem),
        pltpu.async_copy(adj_e_hbm.at[item_v], e1ids, sem),
        pltpu.async_copy(adj_r_hbm.at[item_v], r0buf, sem),
    ]
    for cp in cps:
        cp.wait()
    pltpu.sync_copy(ubuf, u_out.at[pl.ds(base, bpw)])
    pltpu.sync_copy(e0buf, e0_out.at[pl.ds(base, bpw)])

    def p_body(b, _):
        plo = jnp.zeros((16,), jnp.float32)
        phi = jnp.zeros((16,), jnp.float32)
        ulo = ubuf[b, 0:16]
        uhi = ubuf[b, 16:32]
        for d in range(DIM):
            ud = ulo[d] if d < 16 else uhi[d - 16]
            plo = plo + ud * relt_v[d, 0:16]
            phi = phi + ud * relt_v[d, 16:32]
        inv = 1.0 / DIM
        pflat[pl.ds(b * NR, 16)] = plo * inv
        pflat[pl.ds(b * NR + 16, 16)] = phi * inv
        return 0
    lax.fori_loop(0, bpw, p_body, 0)

    def w0_body(b, _):
        rvec = r0buf[b, 0:16]
        score = plsc.load_gather(pflat, [b * NR + rvec])
        e = jnp.exp(score)
        s = jnp.sum(e)
        w0buf[b, 0:16] = e * (1.0 / jnp.broadcast_to(s, (16,)))
        return 0
    lax.fori_loop(0, bpw, w0_body, 0)
    pltpu.sync_copy(w0buf, w0_out.at[pl.ds(base, bpw)])

    n_chunks = bpw // CH_B
    nodes = CH_B * NN

    def chunk_body(c, _):
        for bb in range(CH_B):
            eflat[pl.ds(bb * NN, NN)] = e1ids[c * CH_B + bb, 0:16]
        cps = [
            pltpu.async_copy(adj_e_hbm.at[eflat], nbr, sem),
            pltpu.async_copy(adj_r_hbm.at[eflat], rel2, sem),
            pltpu.async_copy(entity_emb_hbm.at[eflat], e1buf, sem),
        ]
        cps[0].wait()
        cps[1].wait()
        cps[2].wait()
        def flat_body(i, _):
            nbflat[pl.ds(i * NN, NN)] = nbr[i, 0:16]
            return 0
        lax.fori_loop(0, nodes, flat_body, 0)
        cps = [
            pltpu.async_copy(
                entity_emb_hbm.at[nbflat.at[pl.ds(j * 128, 128)]],
                ebuf.at[pl.ds(j * 128, 128)], sem)
            for j in range(nodes * NN // 128)
        ]
        for cp in cps:
            cp.wait()

        def node_body(node, _):
            b_loc = c * CH_B + node // NN
            rvec = rel2[node, 0:16]
            score = plsc.load_gather(pflat, [b_loc * NR + rvec])
            e = jnp.exp(score)
            s = jnp.sum(e)
            alo = jnp.zeros((16,), jnp.float32)
            ahi = jnp.zeros((16,), jnp.float32)
            for n in range(NN):
                wn = e[n]
                alo = alo + wn * ebuf[node * NN + n, 0:16]
                ahi = ahi + wn * ebuf[node * NN + n, 16:32]
            sinv = 1.0 / jnp.broadcast_to(s, (16,))
            aggbuf[node, 0:16] = alo * sinv
            aggbuf[node, 16:32] = ahi * sinv
            return 0
        lax.fori_loop(0, nodes, node_body, 0)

        row0 = (base + c * CH_B) * NN
        pltpu.sync_copy(e1buf, e1_out.at[pl.ds(row0, nodes)])
        pltpu.sync_copy(aggbuf, agg_out.at[pl.ds(row0, nodes)])
        return 0

    lax.fori_loop(0, n_chunks, chunk_body, 0)


def _tc_kernel_body(e1_ref, agg_ref, e0_ref, w0v_ref, u_ref,
                    w0abd_ref, w0bbd_ref, exp_ref, seg_ref,
                    w0a_ref, w0b_ref, w1a_ref, w1b_ref,
                    b0_ref, b0t_ref, b1_ref, out_ref):
    f32 = jnp.float32
    e1 = e1_ref[...]
    agg = agg_ref[...]
    w0v = w0v_ref[...]
    wexp = jnp.dot(w0v, exp_ref[...], preferred_element_type=f32)
    v1 = jnp.maximum(
        jnp.dot(e1, w0abd_ref[...], preferred_element_type=f32)
        + jnp.dot(agg, w0bbd_ref[...], preferred_element_type=f32)
        + b0t_ref[...], 0.0)
    seg = seg_ref[...]
    agg_a = jnp.dot(wexp * e1, seg, preferred_element_type=f32)
    agg_b = jnp.dot(wexp * v1, seg, preferred_element_type=f32)
    h0 = jnp.maximum(
        jnp.dot(e0_ref[...], w0a_ref[...], preferred_element_type=f32)
        + jnp.dot(agg_a, w0b_ref[...], preferred_element_type=f32)
        + b0_ref[...], 0.0)
    out = jnp.tanh(
        jnp.dot(h0, w1a_ref[...], preferred_element_type=f32)
        + jnp.dot(agg_b, w1b_ref[...], preferred_element_type=f32)
        + b1_ref[...])
    out_ref[...] = jax.nn.sigmoid(jnp.sum(out * u_ref[...], axis=1))


def kernel(user_indices, item_indices, adj_entity, adj_relation,
           user_emb, entity_emb, relation_emb, W0, b0, W1, b1):
    B = user_indices.shape[0]
    bpw = B // NW
    rel_t = relation_emb.T

    mesh = plsc.VectorSubcoreMesh(core_axis_name="c", subcore_axis_name="s")
    out_type = (
        jax.ShapeDtypeStruct((B, DIM), jnp.float32),
        jax.ShapeDtypeStruct((B, DIM), jnp.float32),
        jax.ShapeDtypeStruct((B, NN), jnp.float32),
        jax.ShapeDtypeStruct((B * NN, DIM), jnp.float32),
        jax.ShapeDtypeStruct((B * NN, DIM), jnp.float32),
    )
    scratch = [
        pltpu.VMEM((bpw,), jnp.int32),
        pltpu.VMEM((bpw,), jnp.int32),
        pltpu.VMEM((bpw, DIM), jnp.float32),
        pltpu.VMEM((bpw, DIM), jnp.float32),
        pltpu.VMEM((bpw, NN), jnp.int32),
        pltpu.VMEM((bpw, NN), jnp.int32),
        pltpu.VMEM((DIM, NR), jnp.float32),
        pltpu.VMEM((bpw * NR,), jnp.float32),
        pltpu.VMEM((bpw, NN), jnp.float32),
        pltpu.VMEM((CH_B * NN,), jnp.int32),
        pltpu.VMEM((CH_B * NN * NN,), jnp.int32),
        pltpu.VMEM((CH_B * NN, NN), jnp.int32),
        pltpu.VMEM((CH_B * NN, NN), jnp.int32),
        pltpu.VMEM((CH_B * NN * NN, DIM), jnp.float32),
        pltpu.VMEM((CH_B * NN, DIM), jnp.float32),
        pltpu.VMEM((CH_B * NN, DIM), jnp.float32),
        pltpu.SemaphoreType.DMA,
    ]
    sc = pl.kernel(
        functools.partial(_sc_kernel_body, bpw=bpw),
        out_type=out_type, mesh=mesh, scratch_types=scratch,
        compiler_params=pltpu.CompilerParams(
            needs_layout_passes=False, use_tc_tiling_on_sc=False),
    )
    u_g, e0, w0v, e1, agg1 = sc(
        user_indices, item_indices, adj_entity, adj_relation,
        user_emb, entity_emb, rel_t)

    e1f = e1.reshape(B, NN * DIM)
    aggf = agg1.reshape(B, NN * DIM)
    del agg1, e1

    W0a, W0b = W0[:DIM, :], W0[DIM:, :]
    W1a, W1b = W1[:DIM, :], W1[DIM:, :]
    eye16 = jnp.eye(NN, dtype=jnp.float32)
    W0A_bd = jnp.kron(eye16, W0a)
    W0B_bd = jnp.kron(eye16, W0b)
    cols = jnp.arange(NN * DIM, dtype=jnp.int32)
    expand = (cols[None, :] // DIM
              == jnp.arange(NN, dtype=jnp.int32)[:, None]).astype(jnp.float32)
    seg = (cols[:, None] % DIM
           == jnp.arange(DIM, dtype=jnp.int32)[None, :]).astype(jnp.float32)
    b0t = jnp.tile(b0, NN).reshape(1, NN * DIM)

    BB = 512
    grid = (B // BB,)
    full = lambda shape: pl.BlockSpec(shape, lambda i: tuple(0 for _ in shape))
    out = pl.pallas_call(
        _tc_kernel_body,
        grid=grid,
        in_specs=[
            pl.BlockSpec((BB, NN * DIM), lambda i: (i, 0)),
            pl.BlockSpec((BB, NN * DIM), lambda i: (i, 0)),
            pl.BlockSpec((BB, DIM), lambda i: (i, 0)),
            pl.BlockSpec((BB, NN), lambda i: (i, 0)),
            pl.BlockSpec((BB, DIM), lambda i: (i, 0)),
            full((NN * DIM, NN * DIM)),
            full((NN * DIM, NN * DIM)),
            full((NN, NN * DIM)),
            full((NN * DIM, DIM)),
            full((DIM, DIM)),
            full((DIM, DIM)),
            full((DIM, DIM)),
            full((DIM, DIM)),
            full((1, DIM)),
            full((1, NN * DIM)),
            full((1, DIM)),
        ],
        out_specs=pl.BlockSpec((BB,), lambda i: (i,)),
        out_shape=jax.ShapeDtypeStruct((B,), jnp.float32),
    )(e1f, aggf, e0, w0v, u_g, W0A_bd, W0B_bd, expand, seg,
      W0a, W0b, W1a, W1b, b0.reshape(1, DIM), b0t, b1.reshape(1, DIM))
    return out


# speedup vs baseline: 25.0420x; 1.1956x over previous
import functools

import jax
import jax.numpy as jnp
from jax import lax
from jax.experimental import pallas as pl
from jax.experimental.pallas import tpu as pltpu
from jax.experimental.pallas import tpu_sc as plsc

DIM = 32
NN = 16
NR = 32
NW = 32
CH_B = 4


def _sc_kernel_body(
    user_idx_hbm, item_idx_hbm, adj_e_hbm, adj_r_hbm,
    user_emb_hbm, entity_emb_hbm, rel_t_hbm,
    u_out, e0_out, w0_out, e1_out, agg_out,
    item_v, user_v, ubuf, e0buf, e1ids, r0buf, relt_v, pflat, w0buf,
    eflat0, eflat1, nbflat0, nbflat1, nbr0, nbr1, rel20, rel21,
    ebuf0, ebuf1, e1buf0, e1buf1, aggbuf,
    sem, sem_adj0, sem_adj1, sem_e0, sem_e1,
    *, bpw,
):
    cid = lax.axis_index("c")
    sid = lax.axis_index("s")
    wid = sid * 2 + cid
    base = wid * bpw

    pltpu.sync_copy(item_idx_hbm.at[pl.ds(base, bpw)], item_v)
    pltpu.sync_copy(user_idx_hbm.at[pl.ds(base, bpw)], user_v)
    pltpu.sync_copy(rel_t_hbm, relt_v)
    cps = [
        pltpu.async_copy(user_emb_hbm.at[user_v], ubuf, sem),
        pltpu.async_copy(entity_emb_hbm.at[item_v], e0buf, sem),
        pltpu.async_copy(adj_e_hbm.at[item_v], e1ids, sem),
        pltpu.async_copy(adj_r_hbm.at[item_v], r0buf, sem),
    ]
    for cp in cps:
        cp.wait()
    pltpu.sync_copy(ubuf, u_out.at[pl.ds(base, bpw)])
    pltpu.sync_copy(e0buf, e0_out.at[pl.ds(base, bpw)])

    def p_body(b, _):
        plo = jnp.zeros((16,), jnp.float32)
        phi = jnp.zeros((16,), jnp.float32)
        ulo = ubuf[b, 0:16]
        uhi = ubuf[b, 16:32]
        for d in range(DIM):
            ud = ulo[d] if d < 16 else uhi[d - 16]
            plo = plo + ud * relt_v[d, 0:16]
            phi = phi + ud * relt_v[d, 16:32]
        inv = 1.0 / DIM
        pflat[pl.ds(b * NR, 16)] = plo * inv
        pflat[pl.ds(b * NR + 16, 16)] = phi * inv
        return 0
    lax.fori_loop(0, bpw, p_body, 0)

    def w0_body(b, _):
        rvec = r0buf[b, 0:16]
        score = plsc.load_gather(pflat, [b * NR + rvec])
        e = jnp.exp(score)
        s = jnp.sum(e)
        w0buf[b, 0:16] = e * (1.0 / jnp.broadcast_to(s, (16,)))
        return 0
    lax.fori_loop(0, bpw, w0_body, 0)
    pltpu.sync_copy(w0buf, w0_out.at[pl.ds(base, bpw)])

    n_chunks = bpw // CH_B
    nodes = CH_B * NN
    n_estreams = nodes * NN // 128

    eflat_ = (eflat0, eflat1)
    nbr_ = (nbr0, nbr1)
    rel2_ = (rel20, rel21)
    e1buf_ = (e1buf0, e1buf1)
    nbflat_ = (nbflat0, nbflat1)
    ebuf_ = (ebuf0, ebuf1)
    sem_adj_ = (sem_adj0, sem_adj1)
    sem_e_ = (sem_e0, sem_e1)

    def adj_copies(c, s):
        return [
            pltpu.make_async_copy(adj_e_hbm.at[eflat_[s]], nbr_[s],
                                  sem_adj_[s]),
            pltpu.make_async_copy(adj_r_hbm.at[eflat_[s]], rel2_[s],
                                  sem_adj_[s]),
            pltpu.make_async_copy(entity_emb_hbm.at[eflat_[s]], e1buf_[s],
                                  sem_adj_[s]),
        ]

    def fire_adj(c, s):
        for bb in range(CH_B):
            eflat_[s][pl.ds(bb * NN, NN)] = e1ids[c * CH_B + bb, 0:16]
        for cp in adj_copies(c, s):
            cp.start()

    def wait_adj(c, s):
        for cp in adj_copies(c, s):
            cp.wait()
        pltpu.sync_copy(e1buf_[s], e1_out.at[pl.ds((base + c * CH_B) * NN,
                                                   nodes)])

    def e_copies(s):
        return [
            pltpu.make_async_copy(
                entity_emb_hbm.at[nbflat_[s].at[pl.ds(j * 128, 128)]],
                ebuf_[s].at[pl.ds(j * 128, 128)], sem_e_[s])
            for j in range(n_estreams)
        ]

    def fire_e(s):
        def flat_body(i, _):
            nbflat_[s][pl.ds(i * NN, NN)] = nbr_[s][i, 0:16]
            return 0
        lax.fori_loop(0, nodes, flat_body, 0, unroll=4)
        for cp in e_copies(s):
            cp.start()

    def compute(c, s):
        for cp in e_copies(s):
            cp.wait()

        def node_body(node, _):
            b_loc = c * CH_B + node // NN
            rvec = rel2_[s][node, 0:16]
            score = plsc.load_gather(pflat, [b_loc * NR + rvec])
            e = jnp.exp(score)
            ssum = jnp.sum(e)
            alo = jnp.zeros((16,), jnp.float32)
            ahi = jnp.zeros((16,), jnp.float32)
            for n in range(NN):
                wn = e[n]
                alo = alo + wn * ebuf_[s][node * NN + n, 0:16]
                ahi = ahi + wn * ebuf_[s][node * NN + n, 16:32]
            sinv = 1.0 / jnp.broadcast_to(ssum, (16,))
            aggbuf[node, 0:16] = alo * sinv
            aggbuf[node, 16:32] = ahi * sinv
            return 0
        lax.fori_loop(0, nodes, node_body, 0, unroll=2)
        pltpu.sync_copy(aggbuf, agg_out.at[pl.ds((base + c * CH_B) * NN,
                                                 nodes)])

    fire_adj(0, 0)
    wait_adj(0, 0)
    fire_e(0)
    fire_adj(1, 1)

    def pair_body(g, _):
        c0 = 2 * g

        @pl.when(g + 1 < n_chunks // 2)
        def _():
            fire_adj(c0 + 2, 0)
        wait_adj(c0 + 1, 1)
        fire_e(1)
        compute(c0, 0)

        @pl.when(g + 1 < n_chunks // 2)
        def _():
            fire_adj(c0 + 3, 1)

        @pl.when(g + 1 < n_chunks // 2)
        def _():
            wait_adj(c0 + 2, 0)
            fire_e(0)
        compute(c0 + 1, 1)
        return 0

    lax.fori_loop(0, n_chunks // 2, pair_body, 0)


def _tc_kernel_body(e1_ref, agg_ref, e0_ref, w0v_ref, u_ref,
                    w0abd_ref, w0bbd_ref, exp_ref, seg_ref,
                    w0a_ref, w0b_ref, w1a_ref, w1b_ref,
                    b0_ref, b0t_ref, b1_ref, out_ref):
    f32 = jnp.float32
    e1 = e1_ref[...]
    agg = agg_ref[...]
    w0v = w0v_ref[...]
    wexp = jnp.dot(w0v, exp_ref[...], preferred_element_type=f32)
    v1 = jnp.maximum(
        jnp.dot(e1, w0abd_ref[...], preferred_element_type=f32)
        + jnp.dot(agg, w0bbd_ref[...], preferred_element_type=f32)
        + b0t_ref[...], 0.0)
    seg = seg_ref[...]
    agg_a = jnp.dot(wexp * e1, seg, preferred_element_type=f32)
    agg_b = jnp.dot(wexp * v1, seg, preferred_element_type=f32)
    h0 = jnp.maximum(
        jnp.dot(e0_ref[...], w0a_ref[...], preferred_element_type=f32)
        + jnp.dot(agg_a, w0b_ref[...], preferred_element_type=f32)
        + b0_ref[...], 0.0)
    out = jnp.tanh(
        jnp.dot(h0, w1a_ref[...], preferred_element_type=f32)
        + jnp.dot(agg_b, w1b_ref[...], preferred_element_type=f32)
        + b1_ref[...])
    out_ref[...] = jax.nn.sigmoid(jnp.sum(out * u_ref[...], axis=1))


def kernel(user_indices, item_indices, adj_entity, adj_relation,
           user_emb, entity_emb, relation_emb, W0, b0, W1, b1):
    B = user_indices.shape[0]
    bpw = B // NW
    rel_t = relation_emb.T

    mesh = plsc.VectorSubcoreMesh(core_axis_name="c", subcore_axis_name="s")
    out_type = (
        jax.ShapeDtypeStruct((B, DIM), jnp.float32),
        jax.ShapeDtypeStruct((B, DIM), jnp.float32),
        jax.ShapeDtypeStruct((B, NN), jnp.float32),
        jax.ShapeDtypeStruct((B * NN, DIM), jnp.float32),
        jax.ShapeDtypeStruct((B * NN, DIM), jnp.float32),
    )
    scratch = [
        pltpu.VMEM((bpw,), jnp.int32),
        pltpu.VMEM((bpw,), jnp.int32),
        pltpu.VMEM((bpw, DIM), jnp.float32),
        pltpu.VMEM((bpw, DIM), jnp.float32),
        pltpu.VMEM((bpw, NN), jnp.int32),
        pltpu.VMEM((bpw, NN), jnp.int32),
        pltpu.VMEM((DIM, NR), jnp.float32),
        pltpu.VMEM((bpw * NR,), jnp.float32),
        pltpu.VMEM((bpw, NN), jnp.float32),
        pltpu.VMEM((CH_B * NN,), jnp.int32),
        pltpu.VMEM((CH_B * NN,), jnp.int32),
        pltpu.VMEM((CH_B * NN * NN,), jnp.int32),
        pltpu.VMEM((CH_B * NN * NN,), jnp.int32),
        pltpu.VMEM((CH_B * NN, NN), jnp.int32),
        pltpu.VMEM((CH_B * NN, NN), jnp.int32),
        pltpu.VMEM((CH_B * NN, NN), jnp.int32),
        pltpu.VMEM((CH_B * NN, NN), jnp.int32),
        pltpu.VMEM((CH_B * NN * NN, DIM), jnp.float32),
        pltpu.VMEM((CH_B * NN * NN, DIM), jnp.float32),
        pltpu.VMEM((CH_B * NN, DIM), jnp.float32),
        pltpu.VMEM((CH_B * NN, DIM), jnp.float32),
        pltpu.VMEM((CH_B * NN, DIM), jnp.float32),
        pltpu.SemaphoreType.DMA,
        pltpu.SemaphoreType.DMA,
        pltpu.SemaphoreType.DMA,
        pltpu.SemaphoreType.DMA,
        pltpu.SemaphoreType.DMA,
    ]
    sc = pl.kernel(
        functools.partial(_sc_kernel_body, bpw=bpw),
        out_type=out_type, mesh=mesh, scratch_types=scratch,
        compiler_params=pltpu.CompilerParams(
            needs_layout_passes=False, use_tc_tiling_on_sc=False),
    )
    u_g, e0, w0v, e1, agg1 = sc(
        user_indices, item_indices, adj_entity, adj_relation,
        user_emb, entity_emb, rel_t)

    e1f = e1.reshape(B, NN * DIM)
    aggf = agg1.reshape(B, NN * DIM)
    del agg1, e1

    W0a, W0b = W0[:DIM, :], W0[DIM:, :]
    W1a, W1b = W1[:DIM, :], W1[DIM:, :]
    eye16 = jnp.eye(NN, dtype=jnp.float32)
    W0A_bd = jnp.kron(eye16, W0a)
    W0B_bd = jnp.kron(eye16, W0b)
    cols = jnp.arange(NN * DIM, dtype=jnp.int32)
    expand = (cols[None, :] // DIM
              == jnp.arange(NN, dtype=jnp.int32)[:, None]).astype(jnp.float32)
    seg = (cols[:, None] % DIM
           == jnp.arange(DIM, dtype=jnp.int32)[None, :]).astype(jnp.float32)
    b0t = jnp.tile(b0, NN).reshape(1, NN * DIM)

    BB = 512
    grid = (B // BB,)
    full = lambda shape: pl.BlockSpec(shape, lambda i: tuple(0 for _ in shape))
    out = pl.pallas_call(
        _tc_kernel_body,
        grid=grid,
        in_specs=[
            pl.BlockSpec((BB, NN * DIM), lambda i: (i, 0)),
            pl.BlockSpec((BB, NN * DIM), lambda i: (i, 0)),
            pl.BlockSpec((BB, DIM), lambda i: (i, 0)),
            pl.BlockSpec((BB, NN), lambda i: (i, 0)),
            pl.BlockSpec((BB, DIM), lambda i: (i, 0)),
            full((NN * DIM, NN * DIM)),
            full((NN * DIM, NN * DIM)),
            full((NN, NN * DIM)),
            full((NN * DIM, DIM)),
            full((DIM, DIM)),
            full((DIM, DIM)),
            full((DIM, DIM)),
            full((DIM, DIM)),
            full((1, DIM)),
            full((1, NN * DIM)),
            full((1, DIM)),
        ],
        out_specs=pl.BlockSpec((BB,), lambda i: (i,)),
        out_shape=jax.ShapeDtypeStruct((B,), jnp.float32),
    )(e1f, aggf, e0, w0v, u_g, W0A_bd, W0B_bd, expand, seg,
      W0a, W0b, W1a, W1b, b0.reshape(1, DIM), b0t, b1.reshape(1, DIM))
    return out
